# baseline (speedup 1.0000x reference)
import functools

import jax
import jax.numpy as jnp
from jax import lax
from jax.experimental import pallas as pl
from jax.experimental.pallas import tpu as pltpu
from jax.experimental.pallas import tpu_sc as plsc

BATCH = 16384
D = 64
NREL = 1000
NW = 32
BPW = BATCH // NW
CH = 64
NCHUNK = BPW // CH


def _sc_gather_prod(pairs_hbm, ent_h2, ent_t2, out_hbm,
                    pbuf, idx_h, idx_t, row_h, row_t,
                    hh0, tt0, ht0, th0, hh1, tt1, ht1, th1,
                    prod, s0, s1):
    wid = lax.axis_index("s") * 2 + lax.axis_index("c")
    base = wid * BPW
    pltpu.sync_copy(pairs_hbm.at[pl.ds(2 * base, 2 * BPW)], pbuf)
    evens = lax.iota(jnp.int32, 16) * 2
    odds = evens + 1
    for k in range(BPW // 16):
        seg = pbuf.at[pl.ds(32 * k, 32)]
        hv = plsc.load_gather(seg, [evens])
        tv = plsc.load_gather(seg, [odds])
        idx_h[pl.ds(16 * k, 16)] = hv
        idx_t[pl.ds(16 * k, 16)] = tv
        row_h[pl.ds(16 * k, 16)] = hv >> 1
        row_t[pl.ds(16 * k, 16)] = tv >> 1

    bufs = [(hh0, tt0, ht0, th0), (hh1, tt1, ht1, th1)]
    sems = [s0, s1]

    def fire(off, which):
        hh, tt, ht, th = bufs[which]
        sem = sems[which]
        ih = row_h.at[pl.ds(off, CH)]
        it = row_t.at[pl.ds(off, CH)]
        pltpu.async_copy(ent_h2.at[ih], hh, sem)
        pltpu.async_copy(ent_t2.at[it], tt, sem)
        pltpu.async_copy(ent_h2.at[it], ht, sem)
        pltpu.async_copy(ent_t2.at[ih], th, sem)

    def drain(which):
        hh, tt, ht, th = bufs[which]
        sem = sems[which]
        pltpu.make_async_copy(ent_h2.at[pl.ds(0, CH)], hh, sem).wait()
        pltpu.make_async_copy(ent_h2.at[pl.ds(0, CH)], tt, sem).wait()
        pltpu.make_async_copy(ent_h2.at[pl.ds(0, CH)], ht, sem).wait()
        pltpu.make_async_copy(ent_h2.at[pl.ds(0, CH)], th, sem).wait()

    def compute(off, which):
        hh, tt, ht, th = bufs[which]
        for g in range(CH // 16):
            hv = idx_h[pl.ds(off + 16 * g, 16)]
            tv = idx_t[pl.ds(off + 16 * g, 16)]
            for q in range(16):
                r = 16 * g + q
                ho = (hv[q] & 1) * D
                to = (tv[q] & 1) * D
                for j in range(D // 16):
                    hs = pl.ds(ho + 16 * j, 16)
                    ts = pl.ds(to + 16 * j, 16)
                    prod[r, pl.ds(16 * j, 16)] = hh[r, hs] * tt[r, ts]
                    prod[r, pl.ds(D + 16 * j, 16)] = ht[r, ts] * th[r, hs]
        pltpu.sync_copy(prod, out_hbm.at[pl.ds(base + off, CH)])

    fire(0, 0)

    def step(ci, _):
        off0 = pl.multiple_of(2 * ci * CH, CH)
        off1 = pl.multiple_of((2 * ci + 1) * CH, CH)
        off2 = pl.multiple_of((2 * ci + 2) * CH, CH)
        fire(off1, 1)
        drain(0)
        compute(off0, 0)

        @pl.when(ci + 1 < NCHUNK // 2)
        def _():
            fire(off2, 0)

        drain(1)
        compute(off1, 1)
        return 0

    lax.fori_loop(0, NCHUNK // 2, step, 0)


def _tc_score(x_ref, w_ref, o_ref):
    acc = jnp.dot(x_ref[...], w_ref[...], preferred_element_type=jnp.float32)
    o_ref[...] = jnp.clip(acc * 0.5, -20.0, 20.0)


def kernel(pairs, ent_h, ent_t, rel, rel_inv):
    pairs_flat = pairs.astype(jnp.int32).reshape(2 * BATCH)
    n_ent = ent_h.shape[0]
    ent_h2 = ent_h.reshape(n_ent // 2, 2 * D)
    ent_t2 = ent_t.reshape(n_ent // 2, 2 * D)

    mesh = plsc.VectorSubcoreMesh(core_axis_name="c", subcore_axis_name="s")
    sc_fn = functools.partial(
        pl.kernel,
        mesh=mesh,
        out_type=jax.ShapeDtypeStruct((BATCH, 2 * D), jnp.float32),
        scratch_types=[
            pltpu.VMEM((2 * BPW,), jnp.int32),
            pltpu.VMEM((BPW,), jnp.int32),
            pltpu.VMEM((BPW,), jnp.int32),
            pltpu.VMEM((BPW,), jnp.int32),
            pltpu.VMEM((BPW,), jnp.int32),
            pltpu.VMEM((CH, 2 * D), jnp.float32),
            pltpu.VMEM((CH, 2 * D), jnp.float32),
            pltpu.VMEM((CH, 2 * D), jnp.float32),
            pltpu.VMEM((CH, 2 * D), jnp.float32),
            pltpu.VMEM((CH, 2 * D), jnp.float32),
            pltpu.VMEM((CH, 2 * D), jnp.float32),
            pltpu.VMEM((CH, 2 * D), jnp.float32),
            pltpu.VMEM((CH, 2 * D), jnp.float32),
            pltpu.VMEM((CH, 2 * D), jnp.float32),
            pltpu.SemaphoreType.DMA,
            pltpu.SemaphoreType.DMA,
        ],
        compiler_params=pltpu.CompilerParams(
            use_tc_tiling_on_sc=True, needs_layout_passes=False
        ),
    )(_sc_gather_prod)
    prod = sc_fn(pairs_flat, ent_h2, ent_t2)

    w = jnp.concatenate([rel, rel_inv], axis=1).T

    bb = 512
    out = pl.pallas_call(
        _tc_score,
        grid=(BATCH // bb,),
        in_specs=[
            pl.BlockSpec((bb, 2 * D), lambda i: (i, 0)),
            pl.BlockSpec((2 * D, NREL), lambda i: (0, 0)),
        ],
        out_specs=pl.BlockSpec((bb, NREL), lambda i: (i, 0)),
        out_shape=jax.ShapeDtypeStruct((BATCH, NREL), jnp.float32),
    )(prod, w)
    return out


# speedup vs baseline: 1.5090x; 1.5090x over previous
import functools

import jax
import jax.numpy as jnp
from jax import lax
from jax.experimental import pallas as pl
from jax.experimental.pallas import tpu as pltpu
from jax.experimental.pallas import tpu_sc as plsc

BATCH = 16384
D = 64
NREL = 1000
NW = 32
BPW = BATCH // NW
CH = 16
NCHUNK = BPW // CH


def _sc_gather_prod(pairs_hbm, ent_h, ent_t, out_hbm,
                    pbuf, idx_h, idx_t,
                    hh0, tt0, ht0, th0, hh1, tt1, ht1, th1,
                    prod, s_h0, s_t0, s_h1, s_t1):
    wid = lax.axis_index("s") * 2 + lax.axis_index("c")
    base = wid * BPW
    pltpu.sync_copy(pairs_hbm.at[pl.ds(2 * base, 2 * BPW)], pbuf)
    evens = lax.iota(jnp.int32, 16) * 2
    odds = evens + 1
    for k in range(BPW // 16):
        seg = pbuf.at[pl.ds(32 * k, 32)]
        idx_h[pl.ds(16 * k, 16)] = plsc.load_gather(seg, [evens])
        idx_t[pl.ds(16 * k, 16)] = plsc.load_gather(seg, [odds])

    bufs = [(hh0, tt0, ht0, th0), (hh1, tt1, ht1, th1)]
    sems = [(s_h0, s_t0), (s_h1, s_t1)]

    def fire(off, which):
        hh, tt, ht, th = bufs[which]
        s_h, s_t = sems[which]
        hv = idx_h[pl.ds(off, CH)]
        tv = idx_t[pl.ds(off, CH)]
        for r in range(CH):
            hs = hv[r]
            ts = tv[r]
            pltpu.async_copy(ent_h.at[hs], hh.at[r], s_h)
            pltpu.async_copy(ent_t.at[ts], tt.at[r], s_t)
            pltpu.async_copy(ent_h.at[ts], ht.at[r], s_h)
            pltpu.async_copy(ent_t.at[hs], th.at[r], s_t)

    def drain(which):
        hh, tt, ht, th = bufs[which]
        s_h, s_t = sems[which]
        pltpu.make_async_copy(ent_h.at[pl.ds(0, CH)], hh, s_h).wait()
        pltpu.make_async_copy(ent_h.at[pl.ds(0, CH)], ht, s_h).wait()
        pltpu.make_async_copy(ent_t.at[pl.ds(0, CH)], tt, s_t).wait()
        pltpu.make_async_copy(ent_t.at[pl.ds(0, CH)], th, s_t).wait()

    def compute(off, which):
        hh, tt, ht, th = bufs[which]
        for r in range(CH):
            for j in range(D // 16):
                s = pl.ds(16 * j, 16)
                prod[r, pl.ds(16 * j, 16)] = hh[r, s] * tt[r, s]
                prod[r, pl.ds(D + 16 * j, 16)] = ht[r, s] * th[r, s]
        pltpu.sync_copy(prod, out_hbm.at[pl.ds(base + off, CH)])

    fire(0, 0)

    def step(ci, _):
        off0 = pl.multiple_of(2 * ci * CH, CH)
        off1 = pl.multiple_of((2 * ci + 1) * CH, CH)
        off2 = pl.multiple_of((2 * ci + 2) * CH, CH)
        fire(off1, 1)
        drain(0)
        compute(off0, 0)

        @pl.when(ci + 1 < NCHUNK // 2)
        def _():
            fire(off2, 0)

        drain(1)
        compute(off1, 1)
        return 0

    lax.fori_loop(0, NCHUNK // 2, step, 0)


def _tc_score(x_ref, w_ref, o_ref):
    acc = jnp.dot(x_ref[...], w_ref[...], preferred_element_type=jnp.float32)
    o_ref[...] = jnp.clip(acc * 0.5, -20.0, 20.0)


def kernel(pairs, ent_h, ent_t, rel, rel_inv):
    pairs_flat = pairs.astype(jnp.int32).reshape(2 * BATCH)

    mesh = plsc.VectorSubcoreMesh(core_axis_name="c", subcore_axis_name="s")
    sc_fn = functools.partial(
        pl.kernel,
        mesh=mesh,
        out_type=jax.ShapeDtypeStruct((BATCH, 2 * D), jnp.float32),
        scratch_types=[
            pltpu.VMEM((2 * BPW,), jnp.int32),
            pltpu.VMEM((BPW,), jnp.int32),
            pltpu.VMEM((BPW,), jnp.int32),
            pltpu.VMEM((CH, D), jnp.float32),
            pltpu.VMEM((CH, D), jnp.float32),
            pltpu.VMEM((CH, D), jnp.float32),
            pltpu.VMEM((CH, D), jnp.float32),
            pltpu.VMEM((CH, D), jnp.float32),
            pltpu.VMEM((CH, D), jnp.float32),
            pltpu.VMEM((CH, D), jnp.float32),
            pltpu.VMEM((CH, D), jnp.float32),
            pltpu.VMEM((CH, 2 * D), jnp.float32),
            pltpu.SemaphoreType.DMA,
            pltpu.SemaphoreType.DMA,
            pltpu.SemaphoreType.DMA,
            pltpu.SemaphoreType.DMA,
        ],
        compiler_params=pltpu.CompilerParams(
            use_tc_tiling_on_sc=True, needs_layout_passes=False
        ),
    )(_sc_gather_prod)
    prod = sc_fn(pairs_flat, ent_h, ent_t)

    w = jnp.concatenate([rel, rel_inv], axis=1).T

    bb = 512
    out = pl.pallas_call(
        _tc_score,
        grid=(BATCH // bb,),
        in_specs=[
            pl.BlockSpec((bb, 2 * D), lambda i: (i, 0)),
            pl.BlockSpec((2 * D, NREL), lambda i: (0, 0)),
        ],
        out_specs=pl.BlockSpec((bb, NREL), lambda i: (i, 0)),
        out_shape=jax.ShapeDtypeStruct((BATCH, NREL), jnp.float32),
    )(prod, w)
    return out


# speedup vs baseline: 1.6212x; 1.0743x over previous
import functools

import jax
import jax.numpy as jnp
from jax import lax
from jax.experimental import pallas as pl
from jax.experimental.pallas import tpu as pltpu
from jax.experimental.pallas import tpu_sc as plsc

BATCH = 16384
D = 64
NREL = 1000
NW = 32
BPW = BATCH // NW
CH = 16
NCHUNK = BPW // CH


def _sc_gather_prod(pairs_hbm, ent_h, ent_t, out_hbm,
                    pbuf, idx_h, idx_t,
                    hh0, tt0, ht0, th0, hh1, tt1, ht1, th1,
                    prod, s_h0, s_t0, s_h1, s_t1):
    wid = lax.axis_index("s") * 2 + lax.axis_index("c")
    base = wid * BPW
    pltpu.sync_copy(pairs_hbm.at[pl.ds(2 * base, 2 * BPW)], pbuf)
    evens = lax.iota(jnp.int32, 16) * 2
    odds = evens + 1
    for k in range(BPW // 16):
        seg = pbuf.at[pl.ds(32 * k, 32)]
        idx_h[pl.ds(16 * k, 16)] = plsc.load_gather(seg, [evens])
        idx_t[pl.ds(16 * k, 16)] = plsc.load_gather(seg, [odds])

    bufs = [(hh0, tt0, ht0, th0), (hh1, tt1, ht1, th1)]
    sems = [(s_h0, s_t0), (s_h1, s_t1)]

    def fire(off, which):
        hh, tt, ht, th = bufs[which]
        s_h, s_t = sems[which]
        hv = idx_h[pl.ds(off, CH)]
        tv = idx_t[pl.ds(off, CH)]
        for r in range(CH):
            hs = hv[r]
            ts = tv[r]
            pltpu.async_copy(ent_h.at[hs], hh.at[r], s_h)
            pltpu.async_copy(ent_t.at[ts], tt.at[r], s_t)
            pltpu.async_copy(ent_h.at[ts], ht.at[r], s_h)
            pltpu.async_copy(ent_t.at[hs], th.at[r], s_t)

    def drain(which):
        hh, tt, ht, th = bufs[which]
        s_h, s_t = sems[which]
        pltpu.make_async_copy(ent_h.at[pl.ds(0, CH)], hh, s_h).wait()
        pltpu.make_async_copy(ent_h.at[pl.ds(0, CH)], ht, s_h).wait()
        pltpu.make_async_copy(ent_t.at[pl.ds(0, CH)], tt, s_t).wait()
        pltpu.make_async_copy(ent_t.at[pl.ds(0, CH)], th, s_t).wait()

    def compute(off, which):
        hh, tt, ht, th = bufs[which]
        for r in range(CH):
            for j in range(D // 16):
                s = pl.ds(16 * j, 16)
                prod[r, pl.ds(16 * j, 16)] = hh[r, s] * tt[r, s]
                prod[r, pl.ds(D + 16 * j, 16)] = ht[r, s] * th[r, s]
        pltpu.sync_copy(prod, out_hbm.at[pl.ds(base + off, CH)])

    fire(0, 0)

    def step(ci, _):
        off0 = pl.multiple_of(2 * ci * CH, CH)
        off1 = pl.multiple_of((2 * ci + 1) * CH, CH)
        off2 = pl.multiple_of((2 * ci + 2) * CH, CH)
        fire(off1, 1)
        drain(0)
        compute(off0, 0)

        @pl.when(ci + 1 < NCHUNK // 2)
        def _():
            fire(off2, 0)

        drain(1)
        compute(off1, 1)
        return 0

    lax.fori_loop(0, NCHUNK // 2, step, 0)


def _tc_score(x_ref, w_ref, o_ref):
    acc = lax.dot_general(
        w_ref[...], x_ref[...], (((1,), (1,)), ((), ())),
        preferred_element_type=jnp.float32,
    )
    o_ref[...] = jnp.clip(acc * 0.5, -20.0, 20.0)


def kernel(pairs, ent_h, ent_t, rel, rel_inv):
    pairs_flat = pairs.astype(jnp.int32).reshape(2 * BATCH)

    mesh = plsc.VectorSubcoreMesh(core_axis_name="c", subcore_axis_name="s")
    sc_fn = functools.partial(
        pl.kernel,
        mesh=mesh,
        out_type=jax.ShapeDtypeStruct((BATCH, 2 * D), jnp.float32),
        scratch_types=[
            pltpu.VMEM((2 * BPW,), jnp.int32),
            pltpu.VMEM((BPW,), jnp.int32),
            pltpu.VMEM((BPW,), jnp.int32),
            pltpu.VMEM((CH, D), jnp.float32),
            pltpu.VMEM((CH, D), jnp.float32),
            pltpu.VMEM((CH, D), jnp.float32),
            pltpu.VMEM((CH, D), jnp.float32),
            pltpu.VMEM((CH, D), jnp.float32),
            pltpu.VMEM((CH, D), jnp.float32),
            pltpu.VMEM((CH, D), jnp.float32),
            pltpu.VMEM((CH, D), jnp.float32),
            pltpu.VMEM((CH, 2 * D), jnp.float32),
            pltpu.SemaphoreType.DMA,
            pltpu.SemaphoreType.DMA,
            pltpu.SemaphoreType.DMA,
            pltpu.SemaphoreType.DMA,
        ],
        compiler_params=pltpu.CompilerParams(
            use_tc_tiling_on_sc=True, needs_layout_passes=False
        ),
    )(_sc_gather_prod)
    prod = sc_fn(pairs_flat, ent_h, ent_t)

    w = jnp.concatenate([rel, rel_inv], axis=1)

    bb = 512
    out_t = pl.pallas_call(
        _tc_score,
        grid=(BATCH // bb,),
        in_specs=[
            pl.BlockSpec((bb, 2 * D), lambda i: (i, 0)),
            pl.BlockSpec((NREL, 2 * D), lambda i: (0, 0)),
        ],
        out_specs=pl.BlockSpec((NREL, bb), lambda i: (0, i)),
        out_shape=jax.ShapeDtypeStruct((NREL, BATCH), jnp.float32),
    )(prod, w)
    return out_t.T
